# baseline (speedup 1.0000x reference)
import jax
import jax.numpy as jnp
from jax.experimental import pallas as pl
from jax.experimental.pallas import tpu as pltpu

_BT = 512
_NCHAIN = 2
_ZD = 30
_F1 = 36 * 64
_RF1 = 6 * 64
_RF2 = 13 * 32
_RF2P = 512
_F3 = 784
_ROW3 = [0, 2, 4, 6, 8, 10, 12, 1, 3, 5, 7, 9, 11]


def _dense_deconv_mat(s_even, w_packed, *, P, Cin, k, Wout, Q, Cout):
    cpad = w_packed.shape[1] // (k * k)
    w = w_packed.astype(jnp.float32).reshape(Cin, k * k, cpad)[:, :, :Cout]
    s = s_even.astype(jnp.float32)[:, :P]
    taps = []
    for ky in range(k):
        for kx in range(k):
            sh = ky * Wout + kx
            if sh:
                taps.append(jnp.concatenate(
                    [jnp.zeros((sh, P), jnp.float32), s[:Q - sh]], axis=0))
            else:
                taps.append(s)
    s_all = jnp.stack(taps, axis=0)
    d = jnp.einsum('tqp,cto->pcqo', s_all, w)
    return d.reshape(P * Cin, Q * Cout)


def _decoder_body(z_ref, w1_ref, b1_ref, we_ref, wo_ref, b2_ref,
                  w3_ref, b3_ref, o_ref):
    f32, bf16 = jnp.float32, jnp.bfloat16
    hb = _BT // _NCHAIN

    def run(zt):
        y1 = jnp.dot(zt, w1_ref[...], preferred_element_type=f32)
        y1 = y1 + b1_ref[...]
        a1 = jnp.maximum(y1, 0.1 * y1).astype(bf16)

        xs = jnp.concatenate(
            [a1[:, m * _RF1:(m + 1) * _RF1] for m in range(6)], axis=0)
        zrow = jnp.zeros((hb, _RF1), bf16)
        xa = jnp.concatenate([xs, zrow], axis=0)
        xb = jnp.concatenate([zrow, xs], axis=0)
        xe = jnp.concatenate([xa, xb], axis=1)

        ye = jnp.dot(xe, we_ref[...], preferred_element_type=f32)
        yo = jnp.dot(xs, wo_ref[...], preferred_element_type=f32)
        ye = ye + b2_ref[...]
        yo = yo + b2_ref[...]
        ae = jnp.maximum(ye, 0.1 * ye).astype(bf16)
        ao = jnp.maximum(yo, 0.1 * yo).astype(bf16)

        a2 = jnp.concatenate(
            [ae[g * hb:(g + 1) * hb] for g in range(7)]
            + [ao[g * hb:(g + 1) * hb] for g in range(6)], axis=1)
        y3 = jnp.dot(a2, w3_ref[...], preferred_element_type=f32)
        y3 = y3 + b3_ref[...]
        return 0.5 * (jnp.tanh(0.5 * y3) + 1.0)

    for h in range(_NCHAIN):
        o_ref[h * hb:(h + 1) * hb, :] = run(z_ref[h * hb:(h + 1) * hb, :])


def kernel(z, wl, bl, w1, s1, b1, w2, s2, b2, w3, s3, b3):
    f32, bf16 = jnp.float32, jnp.bfloat16

    d1 = _dense_deconv_mat(s1, w1, P=4, Cin=128, k=4, Wout=6, Q=36, Cout=64)
    d2 = _dense_deconv_mat(s2, w2, P=36, Cin=64, k=3, Wout=13, Q=169, Cout=32)
    d3 = _dense_deconv_mat(s3, w3, P=169, Cin=32, k=4, Wout=28, Q=784, Cout=1)

    w1f = wl.astype(f32) @ d1
    b1f = bl.astype(f32) @ d1 + jnp.tile(b1[0, :64], 36)[None]

    r, c = _RF1, _RF2
    pad = _RF2P - _RF2
    w_et = d2[r:2 * r, 2 * c:3 * c]
    w_eb = d2[0:r, 2 * c:3 * c]
    we = jnp.pad(jnp.concatenate([w_et, w_eb], axis=0),
                 ((0, 0), (0, pad))).astype(bf16)
    wo = jnp.pad(d2[r:2 * r, 3 * c:4 * c],
                 ((0, 0), (0, pad))).astype(bf16)
    b2f = jnp.pad(jnp.tile(b2[0, :32], 13), (0, pad))[None].astype(f32)

    w3r = jnp.stack([d3[oy * c:(oy + 1) * c] for oy in _ROW3], axis=0)
    w3r = jnp.pad(w3r, ((0, 0), (0, pad), (0, 0)))
    w3r = w3r.reshape(13 * _RF2P, _F3).astype(bf16)
    b3f = jnp.tile(b3[0, :1], _F3)[None].astype(f32)

    B = z.shape[0]
    nt = (B + _BT - 1) // _BT
    b_pad = nt * _BT
    zf = z.astype(f32)
    if b_pad != B:
        zf = jnp.pad(zf, ((0, b_pad - B), (0, 0)))

    def fixed(i):
        return (0, 0)

    out = pl.pallas_call(
        _decoder_body,
        out_shape=jax.ShapeDtypeStruct((B, _F3), f32),
        grid=(nt,),
        in_specs=[
            pl.BlockSpec((_BT, _ZD), lambda i: (i, 0)),
            pl.BlockSpec((_ZD, _F1), fixed),
            pl.BlockSpec((1, _F1), fixed),
            pl.BlockSpec((2 * _RF1, _RF2P), fixed),
            pl.BlockSpec((_RF1, _RF2P), fixed),
            pl.BlockSpec((1, _RF2P), fixed),
            pl.BlockSpec((13 * _RF2P, _F3), fixed),
            pl.BlockSpec((1, _F3), fixed),
        ],
        out_specs=pl.BlockSpec((_BT, _F3), lambda i: (i, 0)),
        compiler_params=pltpu.CompilerParams(
            dimension_semantics=("parallel",),
            vmem_limit_bytes=64 << 20,
        ),
    )(zf, w1f, b1f, we, wo, b2f, w3r, b3f)

    return out.reshape(B, 1, 28, 28)


# speedup vs baseline: 1.6752x; 1.6752x over previous
import jax
import jax.numpy as jnp
from jax.experimental import pallas as pl
from jax.experimental.pallas import tpu as pltpu

_BT = 512
_NCHAIN = 2
_ZD = 30
_F1 = 36 * 64
_RF1 = 6 * 64
_RF2 = 13 * 32
_F3 = 784
_ROW3 = [0, 2, 4, 6, 8, 10, 12, 1, 3, 5, 7, 9, 11]


def _dense_deconv_mat(s_even, w_packed, *, P, Cin, k, Wout, Q, Cout):
    cpad = w_packed.shape[1] // (k * k)
    w = w_packed.astype(jnp.float32).reshape(Cin, k * k, cpad)[:, :, :Cout]
    s = s_even.astype(jnp.float32)[:, :P]
    taps = []
    for ky in range(k):
        for kx in range(k):
            sh = ky * Wout + kx
            if sh:
                taps.append(jnp.concatenate(
                    [jnp.zeros((sh, P), jnp.float32), s[:Q - sh]], axis=0))
            else:
                taps.append(s)
    s_all = jnp.stack(taps, axis=0)
    d = jnp.einsum('tqp,cto->pcqo', s_all, w)
    return d.reshape(P * Cin, Q * Cout)


def _decoder_body(z_ref, w1_ref, b1_ref, we_ref, wo_ref, b2_ref,
                  w3_ref, b3_ref, o_ref):
    f32, bf16 = jnp.float32, jnp.bfloat16
    hb = _BT // _NCHAIN

    def leaky(y):
        return jnp.maximum(y, 0.1 * y).astype(bf16)

    def run(zt):
        y1 = jnp.dot(w1_ref[...], zt, preferred_element_type=f32)
        a1 = leaky(y1 + b1_ref[...])

        def r1(m):
            return a1[m * _RF1:(m + 1) * _RF1]

        ae = [leaky(jnp.dot(we_ref[:, _RF1:], r1(0),
                            preferred_element_type=f32) + b2_ref[...])]
        for t in range(1, 6):
            ae.append(leaky(jnp.dot(
                we_ref[...], a1[(t - 1) * _RF1:(t + 1) * _RF1],
                preferred_element_type=f32) + b2_ref[...]))
        ae.append(leaky(jnp.dot(we_ref[:, :_RF1], r1(5),
                                preferred_element_type=f32) + b2_ref[...]))
        ao = [leaky(jnp.dot(wo_ref[...], r1(m),
                            preferred_element_type=f32) + b2_ref[...])
              for m in range(6)]

        tops, bots = [], []
        for m in range(13):
            src = ae[m // 2] if m % 2 == 0 else ao[m // 2]
            d = jnp.dot(w3_ref[m], src, preferred_element_type=f32)
            tops.append(d[:56])
            bots.append(d[56:])
        zpair = jnp.zeros((56, hb), f32)
        tt = jnp.concatenate(tops + [zpair], axis=0)
        bb = jnp.concatenate([zpair] + bots, axis=0)
        y3 = tt + bb + b3_ref[0, 0]
        s = 0.5 * (jnp.tanh(0.5 * y3) + 1.0)
        return jnp.transpose(s.astype(bf16)).astype(f32)

    for h in range(_NCHAIN):
        o_ref[h * hb:(h + 1) * hb, :] = run(z_ref[:, h * hb:(h + 1) * hb])


def kernel(z, wl, bl, w1, s1, b1, w2, s2, b2, w3, s3, b3):
    f32, bf16 = jnp.float32, jnp.bfloat16

    d1 = _dense_deconv_mat(s1, w1, P=4, Cin=128, k=4, Wout=6, Q=36, Cout=64)
    d2 = _dense_deconv_mat(s2, w2, P=36, Cin=64, k=3, Wout=13, Q=169, Cout=32)
    d3 = _dense_deconv_mat(s3, w3, P=169, Cin=32, k=4, Wout=28, Q=784, Cout=1)

    w1f = (wl.astype(f32) @ d1).T.astype(bf16)
    b1f = (bl.astype(f32) @ d1
           + jnp.tile(b1[0, :64], 36)[None]).T

    r, c = _RF1, _RF2
    w_et = d2[r:2 * r, 2 * c:3 * c]
    w_eb = d2[0:r, 2 * c:3 * c]
    we = jnp.concatenate([w_eb, w_et], axis=0).T.astype(bf16)
    wo = d2[r:2 * r, 3 * c:4 * c].T.astype(bf16)
    b2f = jnp.tile(b2[0, :32], 13)[:, None].astype(f32)

    w3t = jnp.stack(
        [d3[m * c:(m + 1) * c, 56 * m:56 * m + 112].T for m in range(13)],
        axis=0).astype(bf16)
    b3f = jnp.reshape(b3[0, 0], (1, 1)).astype(f32)

    B = z.shape[0]
    nt = (B + _BT - 1) // _BT
    b_pad = nt * _BT
    zf = z.astype(bf16)
    if b_pad != B:
        zf = jnp.pad(zf, ((0, b_pad - B), (0, 0)))
    zt = zf.T

    def fixed(i):
        return (0, 0)

    out = pl.pallas_call(
        _decoder_body,
        out_shape=jax.ShapeDtypeStruct((B, _F3), f32),
        grid=(nt,),
        in_specs=[
            pl.BlockSpec((_ZD, _BT), lambda i: (0, i)),
            pl.BlockSpec((_F1, _ZD), fixed),
            pl.BlockSpec((_F1, 1), fixed),
            pl.BlockSpec((_RF2, 2 * _RF1), fixed),
            pl.BlockSpec((_RF2, _RF1), fixed),
            pl.BlockSpec((_RF2, 1), fixed),
            pl.BlockSpec((13, 112, _RF2), lambda i: (0, 0, 0)),
            pl.BlockSpec((1, 1), fixed),
        ],
        out_specs=pl.BlockSpec((_BT, _F3), lambda i: (i, 0)),
        compiler_params=pltpu.CompilerParams(
            dimension_semantics=("parallel",),
            vmem_limit_bytes=64 << 20,
        ),
    )(zt, w1f, b1f, we, wo, b2f, w3t, b3f)

    return out.reshape(B, 1, 28, 28)


# speedup vs baseline: 1.7609x; 1.0511x over previous
import numpy as np

import jax
import jax.numpy as jnp
from jax.experimental import pallas as pl
from jax.experimental.pallas import tpu as pltpu

_BT = 512
_NCHAIN = 2
_ZD = 30
_F1 = 36 * 64
_RF1 = 6 * 64
_RF2 = 13 * 32
_F3 = 784


def _col_scatter(k, wout, win):
    s = np.zeros((k, wout, win), np.float32)
    for j in range(win):
        for kx in range(k):
            s[kx, 2 * j + kx, j] = 1.0
    return jnp.asarray(s)


def _dense_deconv_mat(s_even, w_packed, *, P, Cin, k, Wout, Q, Cout):
    cpad = w_packed.shape[1] // (k * k)
    w = w_packed.astype(jnp.float32).reshape(Cin, k * k, cpad)[:, :, :Cout]
    s = s_even.astype(jnp.float32)[:, :P]
    taps = []
    for ky in range(k):
        for kx in range(k):
            sh = ky * Wout + kx
            if sh:
                taps.append(jnp.concatenate(
                    [jnp.zeros((sh, P), jnp.float32), s[:Q - sh]], axis=0))
            else:
                taps.append(s)
    s_all = jnp.stack(taps, axis=0)
    d = jnp.einsum('tqp,cto->pcqo', s_all, w)
    return d.reshape(P * Cin, Q * Cout)


def _decoder_body(z_ref, w1_ref, b1_ref, we_ref, wo_ref, b2_ref,
                  w3_ref, b3_ref, o_ref):
    f32, bf16 = jnp.float32, jnp.bfloat16
    hb = _BT // _NCHAIN

    def leaky(y):
        return jnp.maximum(y, 0.1 * y).astype(bf16)

    def run(zt):
        y1 = jnp.dot(w1_ref[...], zt, preferred_element_type=f32)
        a1 = leaky(y1 + b1_ref[...])

        def r1(m):
            return a1[m * _RF1:(m + 1) * _RF1]

        ae = [leaky(jnp.dot(we_ref[:, _RF1:], r1(0),
                            preferred_element_type=f32) + b2_ref[...])]
        for t in range(1, 6):
            ae.append(leaky(jnp.dot(
                we_ref[...], a1[(t - 1) * _RF1:(t + 1) * _RF1],
                preferred_element_type=f32) + b2_ref[...]))
        ae.append(leaky(jnp.dot(we_ref[:, :_RF1], r1(5),
                                preferred_element_type=f32) + b2_ref[...]))
        ao = [leaky(jnp.dot(wo_ref[...], r1(m),
                            preferred_element_type=f32) + b2_ref[...])
              for m in range(6)]

        tops, bots = [], []
        for m in range(13):
            src = ae[m // 2] if m % 2 == 0 else ao[m // 2]
            d = jnp.dot(w3_ref[...], src, preferred_element_type=f32)
            tops.append(d[:56])
            bots.append(d[56:])
        zpair = jnp.zeros((56, hb), f32)
        tt = jnp.concatenate(tops + [zpair], axis=0)
        bb = jnp.concatenate([zpair] + bots, axis=0)
        y3 = tt + bb + b3_ref[0, 0]
        s = 0.5 * (jnp.tanh(0.5 * y3) + 1.0)
        return jnp.transpose(s.astype(bf16)).astype(f32)

    for h in range(_NCHAIN):
        o_ref[h * hb:(h + 1) * hb, :] = run(z_ref[:, h * hb:(h + 1) * hb])


def kernel(z, wl, bl, w1, s1, b1, w2, s2, b2, w3, s3, b3):
    f32, bf16 = jnp.float32, jnp.bfloat16

    d1 = _dense_deconv_mat(s1, w1, P=4, Cin=128, k=4, Wout=6, Q=36, Cout=64)
    w1f = (wl.astype(f32) @ d1).T.astype(bf16)
    b1f = (bl.astype(f32) @ d1
           + jnp.tile(b1[0, :64], 36)[None]).T

    w2r = w2.astype(f32).reshape(64, 3, 3, 32)
    sx2 = _col_scatter(3, 13, 6)
    bk2 = [jnp.einsum('cko,kxj->xojc', w2r[:, ky], sx2).reshape(_RF2, _RF1)
           for ky in range(3)]
    we = jnp.concatenate([bk2[2], bk2[0]], axis=1).astype(bf16)
    wo = bk2[1].astype(bf16)
    b2f = jnp.tile(b2[0, :32], 13)[:, None].astype(f32)

    w3r4 = w3.astype(f32).reshape(32, 4, 4, 8)[..., 0]
    sx3 = _col_scatter(4, 28, 13)
    w3t = jnp.einsum('cyk,kxj->yxjc', w3r4, sx3).reshape(112, _RF2)
    w3t = w3t.astype(bf16)
    b3f = jnp.reshape(b3[0, 0], (1, 1)).astype(f32)

    B = z.shape[0]
    nt = (B + _BT - 1) // _BT
    b_pad = nt * _BT
    zf = z.astype(bf16)
    if b_pad != B:
        zf = jnp.pad(zf, ((0, b_pad - B), (0, 0)))
    zt = zf.T

    def fixed(i):
        return (0, 0)

    out = pl.pallas_call(
        _decoder_body,
        out_shape=jax.ShapeDtypeStruct((B, _F3), f32),
        grid=(nt,),
        in_specs=[
            pl.BlockSpec((_ZD, _BT), lambda i: (0, i)),
            pl.BlockSpec((_F1, _ZD), fixed),
            pl.BlockSpec((_F1, 1), fixed),
            pl.BlockSpec((_RF2, 2 * _RF1), fixed),
            pl.BlockSpec((_RF2, _RF1), fixed),
            pl.BlockSpec((_RF2, 1), fixed),
            pl.BlockSpec((112, _RF2), fixed),
            pl.BlockSpec((1, 1), fixed),
        ],
        out_specs=pl.BlockSpec((_BT, _F3), lambda i: (i, 0)),
        compiler_params=pltpu.CompilerParams(
            dimension_semantics=("parallel",),
            vmem_limit_bytes=64 << 20,
        ),
    )(zt, w1f, b1f, we, wo, b2f, w3t, b3f)

    return out.reshape(B, 1, 28, 28)
